# baseline (speedup 1.0000x reference)
import jax
import jax.numpy as jnp
from jax.experimental import pallas as pl
from jax.experimental.pallas import tpu as pltpu
from jax.experimental.pallas import tpu_sc as plsc

_B, _S, _D = 4, 8192, 1024
_NUM_TYPES = 8
_BM = 8
_LANES = 16


def kernel(x, type_id, type_embedding):
    xf = x.reshape(_B * _S, _D)
    idx = jnp.asarray(type_id, jnp.int32).reshape(1, 1)
    mesh = plsc.VectorSubcoreMesh(core_axis_name="core",
                                  subcore_axis_name="subcore")

    @pl.kernel(
        out_type=jax.ShapeDtypeStruct((_B * _S, _D), jnp.float32),
        mesh=mesh,
        scratch_types=[
            pltpu.VMEM((1, 1), jnp.int32),
            pltpu.VMEM((1, _D), jnp.float32),
        ],
    )
    def sc_kernel(idx_hbm, table_hbm, x_hbm, o_hbm, idx_vmem, row_vmem):
        pltpu.sync_copy(idx_hbm, idx_vmem)
        pltpu.sync_copy(table_hbm.at[idx_vmem.at[0]], row_vmem)

        def body(in_vmem, out_vmem):
            @pl.loop(0, _BM)
            def _(c0):
                @pl.loop(0, _D, step=_LANES)
                def _(c1):
                    slc = (pl.ds(c0, 1), pl.ds(c1, _LANES))
                    out_vmem.at[*slc][...] = (
                        in_vmem.at[*slc][...]
                        + row_vmem.at[pl.ds(0, 1), pl.ds(c1, _LANES)][...]
                    )

        pltpu.emit_pipeline(
            body,
            grid=((_B * _S) // _BM,),
            in_specs=[pl.BlockSpec((_BM, _D), index_map=lambda i: (i, 0))],
            out_specs=[pl.BlockSpec((_BM, _D), index_map=lambda i: (i, 0))],
            core_axis_name=("core", "subcore"),
            dimension_semantics=(pltpu.PARALLEL,),
        )(x_hbm, o_hbm)

    out = sc_kernel(idx, type_embedding, xf)
    return out.reshape(_B, _S, _D)


# speedup vs baseline: 2.0566x; 2.0566x over previous
import jax
import jax.numpy as jnp
from jax.experimental import pallas as pl
from jax.experimental.pallas import tpu as pltpu
from jax.experimental.pallas import tpu_sc as plsc

_B, _S, _D = 4, 8192, 1024
_NUM_TYPES = 8
_TC_BLOCK = 2048
_TC_ROWS = 26624
_SC_ROWS = _B * _S - _TC_ROWS
_BM = 8
_LANES = 16


def _tc_add_kernel(idx_ref, table_ref, x_ref, o_ref):
    row = table_ref[pl.ds(idx_ref[0], 1), :]
    o_ref[...] = x_ref[...] + row


def _tc_add(idx, table, xf):
    return pl.pallas_call(
        _tc_add_kernel,
        grid_spec=pltpu.PrefetchScalarGridSpec(
            num_scalar_prefetch=1,
            grid=(_TC_ROWS // _TC_BLOCK,),
            in_specs=[
                pl.BlockSpec((_NUM_TYPES, _D), lambda i, idx: (0, 0)),
                pl.BlockSpec((_TC_BLOCK, _D), lambda i, idx: (i, 0)),
            ],
            out_specs=pl.BlockSpec((_TC_BLOCK, _D), lambda i, idx: (i, 0)),
        ),
        out_shape=jax.ShapeDtypeStruct((_TC_ROWS, _D), jnp.float32),
        compiler_params=pltpu.CompilerParams(
            dimension_semantics=("parallel",),
        ),
    )(idx, table, xf)


def _sc_add(idx, table, xf):
    mesh = plsc.VectorSubcoreMesh(core_axis_name="core",
                                  subcore_axis_name="subcore")

    @pl.kernel(
        out_type=jax.ShapeDtypeStruct((_SC_ROWS, _D), jnp.float32),
        mesh=mesh,
        scratch_types=[
            pltpu.VMEM((1, 1), jnp.int32),
            pltpu.VMEM((1, _D), jnp.float32),
        ],
    )
    def sc_kernel(idx_hbm, table_hbm, x_hbm, o_hbm, idx_vmem, row_vmem):
        pltpu.sync_copy(idx_hbm, idx_vmem)
        pltpu.sync_copy(table_hbm.at[idx_vmem.at[0]], row_vmem)

        def body(in_vmem, out_vmem):
            @pl.loop(0, _BM)
            def _(c0):
                @pl.loop(0, _D, step=_LANES)
                def _(c1):
                    slc = (pl.ds(c0, 1), pl.ds(c1, _LANES))
                    out_vmem.at[*slc][...] = (
                        in_vmem.at[*slc][...]
                        + row_vmem.at[pl.ds(0, 1), pl.ds(c1, _LANES)][...]
                    )

        pltpu.emit_pipeline(
            body,
            grid=(_SC_ROWS // _BM,),
            in_specs=[pl.BlockSpec((_BM, _D),
                                   index_map=lambda i: (i + _TC_ROWS // _BM, 0))],
            out_specs=[pl.BlockSpec((_BM, _D), index_map=lambda i: (i, 0))],
            core_axis_name=("core", "subcore"),
            dimension_semantics=(pltpu.PARALLEL,),
        )(x_hbm, o_hbm)

    return sc_kernel(idx, table, xf)


def kernel(x, type_id, type_embedding):
    xf = x.reshape(_B * _S, _D)
    idx = jnp.asarray(type_id, jnp.int32).reshape(1)
    tc_out = _tc_add(idx, type_embedding, xf)
    sc_out = _sc_add(idx.reshape(1, 1), type_embedding, xf)
    out = jnp.concatenate([tc_out, sc_out], axis=0)
    return out.reshape(_B, _S, _D)


# speedup vs baseline: 2.1383x; 1.0398x over previous
import jax
import jax.numpy as jnp
from jax.experimental import pallas as pl
from jax.experimental.pallas import tpu as pltpu
from jax.experimental.pallas import tpu_sc as plsc

_B, _S, _D = 4, 8192, 1024
_NUM_TYPES = 8
_TC_BLOCK = 2048
_TC_ROWS = 28672
_SC_ROWS = _B * _S - _TC_ROWS
_BM = 8
_LANES = 16


def _tc_add_kernel(idx_ref, table_ref, x_ref, o_ref):
    row = table_ref[pl.ds(idx_ref[0], 1), :]
    o_ref[...] = x_ref[...] + row


def _tc_add(idx, table, xf):
    return pl.pallas_call(
        _tc_add_kernel,
        grid_spec=pltpu.PrefetchScalarGridSpec(
            num_scalar_prefetch=1,
            grid=(_TC_ROWS // _TC_BLOCK,),
            in_specs=[
                pl.BlockSpec((_NUM_TYPES, _D), lambda i, idx: (0, 0)),
                pl.BlockSpec((_TC_BLOCK, _D), lambda i, idx: (i, 0)),
            ],
            out_specs=pl.BlockSpec((_TC_BLOCK, _D), lambda i, idx: (i, 0)),
        ),
        out_shape=jax.ShapeDtypeStruct((_TC_ROWS, _D), jnp.float32),
        compiler_params=pltpu.CompilerParams(
            dimension_semantics=("parallel",),
        ),
    )(idx, table, xf)


def _sc_add(idx, table, xf):
    mesh = plsc.VectorSubcoreMesh(core_axis_name="core",
                                  subcore_axis_name="subcore")

    @pl.kernel(
        out_type=jax.ShapeDtypeStruct((_SC_ROWS, _D), jnp.float32),
        mesh=mesh,
        scratch_types=[
            pltpu.VMEM((1, 1), jnp.int32),
            pltpu.VMEM((1, _D), jnp.float32),
        ],
    )
    def sc_kernel(idx_hbm, table_hbm, x_hbm, o_hbm, idx_vmem, row_vmem):
        pltpu.sync_copy(idx_hbm, idx_vmem)
        pltpu.sync_copy(table_hbm.at[idx_vmem.at[0]], row_vmem)

        def body(in_vmem, out_vmem):
            @pl.loop(0, _BM)
            def _(c0):
                @pl.loop(0, _D, step=_LANES)
                def _(c1):
                    slc = (pl.ds(c0, 1), pl.ds(c1, _LANES))
                    out_vmem.at[*slc][...] = (
                        in_vmem.at[*slc][...]
                        + row_vmem.at[pl.ds(0, 1), pl.ds(c1, _LANES)][...]
                    )

        pltpu.emit_pipeline(
            body,
            grid=(_SC_ROWS // _BM,),
            in_specs=[pl.BlockSpec((_BM, _D),
                                   index_map=lambda i: (i + _TC_ROWS // _BM, 0))],
            out_specs=[pl.BlockSpec((_BM, _D), index_map=lambda i: (i, 0))],
            core_axis_name=("core", "subcore"),
            dimension_semantics=(pltpu.PARALLEL,),
        )(x_hbm, o_hbm)

    return sc_kernel(idx, table, xf)


def kernel(x, type_id, type_embedding):
    xf = x.reshape(_B * _S, _D)
    idx = jnp.asarray(type_id, jnp.int32).reshape(1)
    tc_out = _tc_add(idx, type_embedding, xf)
    sc_out = _sc_add(idx.reshape(1, 1), type_embedding, xf)
    out = jnp.concatenate([tc_out, sc_out], axis=0)
    return out.reshape(_B, _S, _D)


# speedup vs baseline: 4.7852x; 2.2378x over previous
import jax
import jax.numpy as jnp
from jax.experimental import pallas as pl
from jax.experimental.pallas import tpu as pltpu

_B, _S, _D = 4, 8192, 1024
_NUM_TYPES = 8
_BLOCK = 2048


def _add_kernel(idx_ref, table_ref, x_ref, o_ref):
    row = table_ref[pl.ds(idx_ref[0], 1), :]
    o_ref[...] = x_ref[...] + row


def kernel(x, type_id, type_embedding):
    xf = x.reshape(_B * _S, _D)
    idx = jnp.asarray(type_id, jnp.int32).reshape(1)
    grid = (_B * _S // _BLOCK,)
    out = pl.pallas_call(
        _add_kernel,
        grid_spec=pltpu.PrefetchScalarGridSpec(
            num_scalar_prefetch=1,
            grid=grid,
            in_specs=[
                pl.BlockSpec((_NUM_TYPES, _D), lambda i, idx: (0, 0)),
                pl.BlockSpec((_BLOCK, _D), lambda i, idx: (i, 0)),
            ],
            out_specs=pl.BlockSpec((_BLOCK, _D), lambda i, idx: (i, 0)),
        ),
        out_shape=jax.ShapeDtypeStruct((_B * _S, _D), jnp.float32),
        compiler_params=pltpu.CompilerParams(
            dimension_semantics=("parallel",),
        ),
    )(idx, type_embedding, xf)
    return out.reshape(_B, _S, _D)


# speedup vs baseline: 4.7865x; 1.0003x over previous
import jax
import jax.numpy as jnp
from jax.experimental import pallas as pl
from jax.experimental.pallas import tpu as pltpu

_B, _S, _D = 4, 8192, 1024
_NUM_TYPES = 8
_BLOCK = 2048


def _add_kernel(idx_ref, table_ref, x_ref, o_ref):
    row = table_ref[pl.ds(idx_ref[0], 1), :]
    o_ref[...] = x_ref[...] + row


def kernel(x, type_id, type_embedding):
    xf = x.reshape(_B * _S, _D)
    idx = jnp.asarray(type_id, jnp.int32).reshape(1)
    grid = (_B * _S // _BLOCK,)
    out = pl.pallas_call(
        _add_kernel,
        grid_spec=pltpu.PrefetchScalarGridSpec(
            num_scalar_prefetch=1,
            grid=grid,
            in_specs=[
                pl.BlockSpec((_NUM_TYPES, _D), lambda i, idx: (0, 0)),
                pl.BlockSpec((_BLOCK, _D), lambda i, idx: (i, 0)),
            ],
            out_specs=pl.BlockSpec((_BLOCK, _D), lambda i, idx: (i, 0)),
        ),
        out_shape=jax.ShapeDtypeStruct((_B * _S, _D), jnp.float32),
        compiler_params=pltpu.CompilerParams(
            dimension_semantics=("arbitrary",),
        ),
    )(idx, type_embedding, xf)
    return out.reshape(_B, _S, _D)
